# baseline (speedup 1.0000x reference)
import functools

import jax
import jax.numpy as jnp
from jax import lax
from jax.experimental import pallas as pl
from jax.experimental.pallas import tpu as pltpu
from jax.experimental.pallas import tpu_sc as plsc

B = 8
N = 32768
K = 2048
C = 128
L = 16
NCORE = 2
NSUB = 16
MEMBERS = 4
NPER = N // MEMBERS
NCHUNK = NPER // L
IDX_CHUNKS = K // 128
CPS = C // (32 // B)

def _lanes():
    return lax.iota(jnp.int32, L)


def _ext_f32(v, lane):
    return jnp.max(jnp.where(_lanes() == lane, v, -jnp.inf))


def _fps_body(x_hbm, y_hbm, z_hbm, p0_hbm,
              idx_hbm, nx_hbm, ny_hbm, nz_hbm,
              x_v, y_v, z_v, d_v, p0_v, stage_v, gath_v,
              idxb_v, nxb_v, nyb_v, nzb_v, shared):
    core = lax.axis_index("c")
    sub = lax.axis_index("s")
    batch = core * (NSUB // MEMBERS) + sub // MEMBERS
    member = sub % MEMBERS
    gbase = (sub // MEMBERS) * MEMBERS
    base = member * NPER

    pltpu.sync_copy(x_hbm.at[batch, pl.ds(base, NPER)], x_v)
    pltpu.sync_copy(y_hbm.at[batch, pl.ds(base, NPER)], y_v)
    pltpu.sync_copy(z_hbm.at[batch, pl.ds(base, NPER)], z_v)
    pltpu.sync_copy(p0_hbm.at[batch], p0_v)

    inf_v = jnp.full((L,), jnp.inf, dtype=jnp.float32)

    def init_chunk(i, _):
        d_v[pl.ds(i * L, L)] = inf_v
        return 0

    lax.fori_loop(0, NCHUNK, init_chunk, 0)

    p0 = p0_v[...]
    px0 = _ext_f32(p0, 0)
    py0 = _ext_f32(p0, 1)
    pz0 = _ext_f32(p0, 2)

    lane = _lanes()
    acc0_i = jnp.zeros((L,), dtype=jnp.int32)
    acc0_x = jnp.where(lane == 0, px0, 0.0)
    acc0_y = jnp.where(lane == 0, py0, 0.0)
    acc0_z = jnp.where(lane == 0, pz0, 0.0)

    UNROLL = 8

    def step(s, carry):
        px, py, pz, acc_i, acc_x, acc_y, acc_z = carry

        def chunk(i, accs):
            outs = []
            for u in range(UNROLL):
                mv, mi = accs[2 * u], accs[2 * u + 1]
                ci = i * UNROLL + u
                sl = pl.ds(ci * L, L)
                dx = x_v[sl] - px
                dy = y_v[sl] - py
                dz = z_v[sl] - pz
                dist = dx * dx + dy * dy + dz * dz
                dn = jnp.minimum(d_v[sl], dist)
                d_v[sl] = dn
                sel = dn > mv
                idxs = ci * L + _lanes()
                outs.append(jnp.where(sel, dn, mv))
                outs.append(jnp.where(sel, idxs, mi))
            return tuple(outs)

        init = (jnp.full((L,), -jnp.inf, dtype=jnp.float32),
                jnp.zeros((L,), dtype=jnp.int32)) * UNROLL
        accs = lax.fori_loop(0, NCHUNK // UNROLL, chunk, init)

        maxv, maxi = accs[0], accs[1]
        for u in range(1, UNROLL):
            cv, cidx = accs[2 * u], accs[2 * u + 1]
            take = (cv > maxv) | ((cv == maxv) & (cidx < maxi))
            maxv = jnp.where(take, cv, maxv)
            maxi = jnp.where(take, cidx, maxi)

        m = jnp.max(maxv)
        li = jnp.min(jnp.where(maxv == m, maxi, jnp.int32(2147483647)))
        gidx = base + li
        liv = jnp.full((L,), li, dtype=jnp.int32)
        cx = jnp.max(plsc.load_gather(x_v, [liv]))
        cy = jnp.max(plsc.load_gather(y_v, [liv]))
        cz = jnp.max(plsc.load_gather(z_v, [liv]))

        lane = _lanes()
        sv = jnp.where(
            lane == 0, m,
            jnp.where(lane == 1, gidx.astype(jnp.float32),
                      jnp.where(lane == 2, cx,
                                jnp.where(lane == 3, cy,
                                          jnp.where(lane == 4, cz,
                                                    jnp.float32(0.0))))))
        stage_v[...] = sv
        parity = lax.rem(s, 2)
        pltpu.sync_copy(stage_v, shared.at[parity, sub])
        plsc.subcore_barrier()
        pltpu.sync_copy(shared.at[parity, pl.ds(gbase, MEMBERS)], gath_v)

        best = gath_v[0, :]
        for k in range(1, MEMBERS):
            cand = gath_v[k, :]
            bd = _ext_f32(best, 0)
            cd = _ext_f32(cand, 0)
            bi = _ext_f32(best, 1)
            ci = _ext_f32(cand, 1)
            take = (cd > bd) | ((cd == bd) & (ci < bi))
            best = jnp.where(take, cand, best)

        wi = _ext_f32(best, 1).astype(jnp.int32)
        wx = _ext_f32(best, 2)
        wy = _ext_f32(best, 3)
        wz = _ext_f32(best, 4)

        lpos = lax.rem(s, L)
        acc_i = jnp.where(lane == lpos, wi, acc_i)
        acc_x = jnp.where(lane == lpos, wx, acc_x)
        acc_y = jnp.where(lane == lpos, wy, acc_y)
        acc_z = jnp.where(lane == lpos, wz, acc_z)

        @pl.when(jnp.logical_and(member == 0, lpos == L - 1))
        def _():
            cbase = (s // L) * L
            idxb_v[pl.ds(cbase, L)] = acc_i
            nxb_v[pl.ds(cbase, L)] = acc_x
            nyb_v[pl.ds(cbase, L)] = acc_y
            nzb_v[pl.ds(cbase, L)] = acc_z

        return (wx, wy, wz, acc_i, acc_x, acc_y, acc_z)

    lax.fori_loop(1, K, step,
                  (px0, py0, pz0, acc0_i, acc0_x, acc0_y, acc0_z))

    @pl.when(member == 0)
    def _():
        pltpu.sync_copy(idxb_v, idx_hbm.at[batch])
        pltpu.sync_copy(nxb_v, nx_hbm.at[batch])
        pltpu.sync_copy(nyb_v, ny_hbm.at[batch])
        pltpu.sync_copy(nzb_v, nz_hbm.at[batch])


def _gather_body(f_hbm, idx_hbm, out_hbm,
                 idx_v, rbase_v, lsel_v, rows_v, orow_v, sem):
    core = lax.axis_index("c")
    sub = lax.axis_index("s")
    w = core * NSUB + sub
    batch = w // (32 // B)
    cbase = (w % (32 // B)) * CPS

    pltpu.sync_copy(idx_hbm.at[batch], idx_v)

    for j in range(IDX_CHUNKS):
        for l in range(8):
            v = idx_v[pl.ds(j * 128 + l * L, L)]
            rbase_v[j, pl.ds(l * L, L)] = lax.shift_right_logical(v, 4)
            lsel_v[pl.ds(j * 128 + l * L, L)] = lax.bitwise_and(v, 15)

    def chan(ci, _):
        c = cbase + ci
        table = f_hbm.at[batch, c]
        copies = [
            pltpu.async_copy(table.at[rbase_v.at[j]],
                             rows_v.at[pl.ds(j * 128, 128)], sem)
            for j in range(IDX_CHUNKS)
        ]
        for cp in copies:
            cp.wait()

        def extract(k, _):
            rowv = k * L + _lanes()
            lanev = lsel_v[pl.ds(k * L, L)]
            orow_v[pl.ds(k * L, L)] = plsc.load_gather(rows_v, [rowv, lanev])
            return 0

        lax.fori_loop(0, K // L, extract, 0)
        pltpu.sync_copy(orow_v, out_hbm.at[batch, c])
        return 0

    lax.fori_loop(0, CPS, chan, 0)


@jax.jit
def kernel(points_xyz, features):
    mesh = plsc.VectorSubcoreMesh(
        core_axis_name="c", subcore_axis_name="s",
        num_cores=NCORE, num_subcores=NSUB)

    x = points_xyz[:, :, 0]
    y = points_xyz[:, :, 1]
    z = points_xyz[:, :, 2]
    p0 = jnp.pad(points_xyz[:, 0, :], ((0, 0), (0, L - 3)))

    fps = pl.kernel(
        _fps_body,
        out_type=(
            jax.ShapeDtypeStruct((B, K), jnp.int32),
            jax.ShapeDtypeStruct((B, K), jnp.float32),
            jax.ShapeDtypeStruct((B, K), jnp.float32),
            jax.ShapeDtypeStruct((B, K), jnp.float32),
        ),
        mesh=mesh,
        scratch_types=[
            pltpu.VMEM((NPER,), jnp.float32),
            pltpu.VMEM((NPER,), jnp.float32),
            pltpu.VMEM((NPER,), jnp.float32),
            pltpu.VMEM((NPER,), jnp.float32),
            pltpu.VMEM((L,), jnp.float32),
            pltpu.VMEM((L,), jnp.float32),
            pltpu.VMEM((MEMBERS, L), jnp.float32),
            pltpu.VMEM((K,), jnp.int32),
            pltpu.VMEM((K,), jnp.float32),
            pltpu.VMEM((K,), jnp.float32),
            pltpu.VMEM((K,), jnp.float32),
            pltpu.VMEM_SHARED((2, NSUB, L), jnp.float32),
        ],
        compiler_params=pltpu.CompilerParams(
            needs_layout_passes=False, use_tc_tiling_on_sc=False),
        name="fps_sc",
    )
    indices, nx, ny, nz = fps(x, y, z, p0)

    f4 = features.reshape(B, C, K, L)
    gather = pl.kernel(
        _gather_body,
        out_type=jax.ShapeDtypeStruct((B, C, K), jnp.float32),
        mesh=mesh,
        scratch_types=[
            pltpu.VMEM((K,), jnp.int32),
            pltpu.VMEM((IDX_CHUNKS, 128), jnp.int32),
            pltpu.VMEM((K,), jnp.int32),
            pltpu.VMEM((K, L), jnp.float32),
            pltpu.VMEM((K,), jnp.float32),
            pltpu.SemaphoreType.DMA,
        ],
        compiler_params=pltpu.CompilerParams(
            needs_layout_passes=False, use_tc_tiling_on_sc=False),
        name="feat_gather_sc",
    )
    new_fea = gather(f4, indices)

    new_xyz = jnp.stack([nx, ny, nz], axis=-1)
    return new_xyz, new_fea, indices


# speedup vs baseline: 2.3527x; 2.3527x over previous
import functools

import jax
import jax.numpy as jnp
from jax import lax
from jax.experimental import pallas as pl
from jax.experimental.pallas import tpu as pltpu
from jax.experimental.pallas import tpu_sc as plsc

B = 8
N = 32768
K = 2048
C = 128
L = 16
NCORE = 2
NSUB = 16
MEMBERS = 4
NPER = N // MEMBERS
NCHUNK = NPER // L
IDX_CHUNKS = K // 128
CPS = C // (32 // B)

def _lanes():
    return lax.iota(jnp.int32, L)


def _ext_f32(v, lane):
    return jnp.max(jnp.where(_lanes() == lane, v, -jnp.inf))


def _fps_body(x_hbm, y_hbm, z_hbm, p0_hbm,
              idx_hbm, nx_hbm, ny_hbm, nz_hbm,
              x_v, y_v, z_v, d_v, p0_v, stage_v, gath_v,
              idxb_v, nxb_v, nyb_v, nzb_v, shared):
    core = lax.axis_index("c")
    sub = lax.axis_index("s")
    batch = core * (NSUB // MEMBERS) + sub // MEMBERS
    member = sub % MEMBERS
    gbase = (sub // MEMBERS) * MEMBERS
    base = member * NPER

    pltpu.sync_copy(x_hbm.at[batch, pl.ds(base, NPER)], x_v)
    pltpu.sync_copy(y_hbm.at[batch, pl.ds(base, NPER)], y_v)
    pltpu.sync_copy(z_hbm.at[batch, pl.ds(base, NPER)], z_v)
    pltpu.sync_copy(p0_hbm.at[batch], p0_v)

    inf_v = jnp.full((L,), jnp.inf, dtype=jnp.float32)

    def init_chunk(i, _):
        d_v[pl.ds(i * L, L)] = inf_v
        return 0

    lax.fori_loop(0, NCHUNK, init_chunk, 0)

    p0 = p0_v[...]
    px0 = _ext_f32(p0, 0)
    py0 = _ext_f32(p0, 1)
    pz0 = _ext_f32(p0, 2)

    lane = _lanes()
    acc0_i = jnp.zeros((L,), dtype=jnp.int32)
    acc0_x = jnp.where(lane == 0, px0, 0.0)
    acc0_y = jnp.where(lane == 0, py0, 0.0)
    acc0_z = jnp.where(lane == 0, pz0, 0.0)

    UNROLL = 8

    def step(s, carry):
        px, py, pz, acc_i, acc_x, acc_y, acc_z = carry

        def chunk(i, accs):
            mv, mi = accs[0], accs[1]
            sl = pl.ds(i * L, L)
            dx = x_v[sl] - px
            dy = y_v[sl] - py
            dz = z_v[sl] - pz
            dist = dx * dx + dy * dy + dz * dz
            dn = jnp.minimum(d_v[sl], dist)
            d_v[sl] = dn
            idxs = i * L + _lanes()
            sel = (dn > mv) | ((dn == mv) & (idxs < mi))
            nv = jnp.where(sel, dn, mv)
            ni = jnp.where(sel, idxs, mi)
            return accs[2:] + (nv, ni)

        init = (jnp.full((L,), -jnp.inf, dtype=jnp.float32),
                jnp.zeros((L,), dtype=jnp.int32)) * UNROLL
        accs = plsc.parallel_loop(0, NCHUNK, 1, unroll=UNROLL,
                                  carry=init)(chunk)

        maxv, maxi = accs[0], accs[1]
        for u in range(1, UNROLL):
            cv, cidx = accs[2 * u], accs[2 * u + 1]
            take = (cv > maxv) | ((cv == maxv) & (cidx < maxi))
            maxv = jnp.where(take, cv, maxv)
            maxi = jnp.where(take, cidx, maxi)

        m = jnp.max(maxv)
        li = jnp.min(jnp.where(maxv == m, maxi, jnp.int32(2147483647)))
        gidx = base + li
        liv = jnp.full((L,), li, dtype=jnp.int32)
        cx = jnp.max(plsc.load_gather(x_v, [liv]))
        cy = jnp.max(plsc.load_gather(y_v, [liv]))
        cz = jnp.max(plsc.load_gather(z_v, [liv]))

        lane = _lanes()
        sv = jnp.where(
            lane == 0, m,
            jnp.where(lane == 1, gidx.astype(jnp.float32),
                      jnp.where(lane == 2, cx,
                                jnp.where(lane == 3, cy,
                                          jnp.where(lane == 4, cz,
                                                    jnp.float32(0.0))))))
        stage_v[...] = sv
        parity = lax.rem(s, 2)
        pltpu.sync_copy(stage_v, shared.at[parity, sub])
        plsc.subcore_barrier()
        pltpu.sync_copy(shared.at[parity, pl.ds(gbase, MEMBERS)], gath_v)

        best = gath_v[0, :]
        for k in range(1, MEMBERS):
            cand = gath_v[k, :]
            bd = _ext_f32(best, 0)
            cd = _ext_f32(cand, 0)
            bi = _ext_f32(best, 1)
            ci = _ext_f32(cand, 1)
            take = (cd > bd) | ((cd == bd) & (ci < bi))
            best = jnp.where(take, cand, best)

        wi = _ext_f32(best, 1).astype(jnp.int32)
        wx = _ext_f32(best, 2)
        wy = _ext_f32(best, 3)
        wz = _ext_f32(best, 4)

        lpos = lax.rem(s, L)
        acc_i = jnp.where(lane == lpos, wi, acc_i)
        acc_x = jnp.where(lane == lpos, wx, acc_x)
        acc_y = jnp.where(lane == lpos, wy, acc_y)
        acc_z = jnp.where(lane == lpos, wz, acc_z)

        @pl.when(jnp.logical_and(member == 0, lpos == L - 1))
        def _():
            cbase = (s // L) * L
            idxb_v[pl.ds(cbase, L)] = acc_i
            nxb_v[pl.ds(cbase, L)] = acc_x
            nyb_v[pl.ds(cbase, L)] = acc_y
            nzb_v[pl.ds(cbase, L)] = acc_z

        return (wx, wy, wz, acc_i, acc_x, acc_y, acc_z)

    lax.fori_loop(1, K, step,
                  (px0, py0, pz0, acc0_i, acc0_x, acc0_y, acc0_z))

    @pl.when(member == 0)
    def _():
        pltpu.sync_copy(idxb_v, idx_hbm.at[batch])
        pltpu.sync_copy(nxb_v, nx_hbm.at[batch])
        pltpu.sync_copy(nyb_v, ny_hbm.at[batch])
        pltpu.sync_copy(nzb_v, nz_hbm.at[batch])


def _gather_body(f_hbm, idx_hbm, out_hbm,
                 idx_v, rbase_v, lsel_v, rows_v, orow_v, sem):
    core = lax.axis_index("c")
    sub = lax.axis_index("s")
    w = core * NSUB + sub
    batch = w // (32 // B)
    cbase = (w % (32 // B)) * CPS

    pltpu.sync_copy(idx_hbm.at[batch], idx_v)

    for j in range(IDX_CHUNKS):
        for l in range(8):
            v = idx_v[pl.ds(j * 128 + l * L, L)]
            rbase_v[j, pl.ds(l * L, L)] = lax.shift_right_logical(v, 4)
            lsel_v[pl.ds(j * 128 + l * L, L)] = lax.bitwise_and(v, 15)

    def chan(ci, _):
        c = cbase + ci
        table = f_hbm.at[batch, c]
        copies = [
            pltpu.async_copy(table.at[rbase_v.at[j]],
                             rows_v.at[pl.ds(j * 128, 128)], sem)
            for j in range(IDX_CHUNKS)
        ]
        for cp in copies:
            cp.wait()

        def extract(k, _):
            rowv = k * L + _lanes()
            lanev = lsel_v[pl.ds(k * L, L)]
            orow_v[pl.ds(k * L, L)] = plsc.load_gather(rows_v, [rowv, lanev])
            return 0

        lax.fori_loop(0, K // L, extract, 0)
        pltpu.sync_copy(orow_v, out_hbm.at[batch, c])
        return 0

    lax.fori_loop(0, CPS, chan, 0)


@jax.jit
def kernel(points_xyz, features):
    mesh = plsc.VectorSubcoreMesh(
        core_axis_name="c", subcore_axis_name="s",
        num_cores=NCORE, num_subcores=NSUB)

    x = points_xyz[:, :, 0]
    y = points_xyz[:, :, 1]
    z = points_xyz[:, :, 2]
    p0 = jnp.pad(points_xyz[:, 0, :], ((0, 0), (0, L - 3)))

    fps = pl.kernel(
        _fps_body,
        out_type=(
            jax.ShapeDtypeStruct((B, K), jnp.int32),
            jax.ShapeDtypeStruct((B, K), jnp.float32),
            jax.ShapeDtypeStruct((B, K), jnp.float32),
            jax.ShapeDtypeStruct((B, K), jnp.float32),
        ),
        mesh=mesh,
        scratch_types=[
            pltpu.VMEM((NPER,), jnp.float32),
            pltpu.VMEM((NPER,), jnp.float32),
            pltpu.VMEM((NPER,), jnp.float32),
            pltpu.VMEM((NPER,), jnp.float32),
            pltpu.VMEM((L,), jnp.float32),
            pltpu.VMEM((L,), jnp.float32),
            pltpu.VMEM((MEMBERS, L), jnp.float32),
            pltpu.VMEM((K,), jnp.int32),
            pltpu.VMEM((K,), jnp.float32),
            pltpu.VMEM((K,), jnp.float32),
            pltpu.VMEM((K,), jnp.float32),
            pltpu.VMEM_SHARED((2, NSUB, L), jnp.float32),
        ],
        compiler_params=pltpu.CompilerParams(
            needs_layout_passes=False, use_tc_tiling_on_sc=False),
        name="fps_sc",
    )
    indices, nx, ny, nz = fps(x, y, z, p0)

    f4 = features.reshape(B, C, K, L)
    gather = pl.kernel(
        _gather_body,
        out_type=jax.ShapeDtypeStruct((B, C, K), jnp.float32),
        mesh=mesh,
        scratch_types=[
            pltpu.VMEM((K,), jnp.int32),
            pltpu.VMEM((IDX_CHUNKS, 128), jnp.int32),
            pltpu.VMEM((K,), jnp.int32),
            pltpu.VMEM((K, L), jnp.float32),
            pltpu.VMEM((K,), jnp.float32),
            pltpu.SemaphoreType.DMA,
        ],
        compiler_params=pltpu.CompilerParams(
            needs_layout_passes=False, use_tc_tiling_on_sc=False),
        name="feat_gather_sc",
    )
    new_fea = gather(f4, indices)

    new_xyz = jnp.stack([nx, ny, nz], axis=-1)
    return new_xyz, new_fea, indices


# speedup vs baseline: 2.3795x; 1.0114x over previous
import functools

import jax
import jax.numpy as jnp
from jax import lax
from jax.experimental import pallas as pl
from jax.experimental.pallas import tpu as pltpu
from jax.experimental.pallas import tpu_sc as plsc

B = 8
N = 32768
K = 2048
C = 128
L = 16
NCORE = 2
NSUB = 16
MEMBERS = 4
NPER = N // MEMBERS
NCHUNK = NPER // L
IDX_CHUNKS = K // 128
CPS = C // (32 // B)

def _lanes():
    return lax.iota(jnp.int32, L)


def _ext_f32(v, lane):
    return v[lane]


def _fps_body(x_hbm, y_hbm, z_hbm, p0_hbm,
              idx_hbm, nx_hbm, ny_hbm, nz_hbm,
              x_v, y_v, z_v, d_v, p0_v, stage_v, gath_v,
              idxb_v, nxb_v, nyb_v, nzb_v, shared):
    core = lax.axis_index("c")
    sub = lax.axis_index("s")
    batch = core * (NSUB // MEMBERS) + sub // MEMBERS
    member = sub % MEMBERS
    gbase = (sub // MEMBERS) * MEMBERS
    base = member * NPER

    pltpu.sync_copy(x_hbm.at[batch, pl.ds(base, NPER)], x_v)
    pltpu.sync_copy(y_hbm.at[batch, pl.ds(base, NPER)], y_v)
    pltpu.sync_copy(z_hbm.at[batch, pl.ds(base, NPER)], z_v)
    pltpu.sync_copy(p0_hbm.at[batch], p0_v)

    inf_v = jnp.full((L,), jnp.inf, dtype=jnp.float32)

    def init_chunk(i, _):
        d_v[pl.ds(i * L, L)] = inf_v
        return 0

    lax.fori_loop(0, NCHUNK, init_chunk, 0)

    p0 = p0_v[...]
    px0 = _ext_f32(p0, 0)
    py0 = _ext_f32(p0, 1)
    pz0 = _ext_f32(p0, 2)

    lane = _lanes()
    acc0_i = jnp.zeros((L,), dtype=jnp.int32)
    acc0_x = jnp.where(lane == 0, px0, 0.0)
    acc0_y = jnp.where(lane == 0, py0, 0.0)
    acc0_z = jnp.where(lane == 0, pz0, 0.0)

    UNROLL = 8

    def step(s, carry):
        px, py, pz, acc_i, acc_x, acc_y, acc_z = carry

        def chunk(i, accs):
            mv, mi = accs[0], accs[1]
            sl = pl.ds(i * L, L)
            dx = x_v[sl] - px
            dy = y_v[sl] - py
            dz = z_v[sl] - pz
            dist = dx * dx + dy * dy + dz * dz
            dn = jnp.minimum(d_v[sl], dist)
            d_v[sl] = dn
            idxs = i * L + _lanes()
            sel = (dn > mv) | ((dn == mv) & (idxs < mi))
            nv = jnp.where(sel, dn, mv)
            ni = jnp.where(sel, idxs, mi)
            return accs[2:] + (nv, ni)

        init = (jnp.full((L,), -jnp.inf, dtype=jnp.float32),
                jnp.zeros((L,), dtype=jnp.int32)) * UNROLL
        accs = plsc.parallel_loop(0, NCHUNK, 1, unroll=UNROLL,
                                  carry=init)(chunk)

        maxv, maxi = accs[0], accs[1]
        for u in range(1, UNROLL):
            cv, cidx = accs[2 * u], accs[2 * u + 1]
            take = (cv > maxv) | ((cv == maxv) & (cidx < maxi))
            maxv = jnp.where(take, cv, maxv)
            maxi = jnp.where(take, cidx, maxi)

        m = jnp.max(maxv)
        li = jnp.min(jnp.where(maxv == m, maxi, jnp.int32(2147483647)))
        gidx = base + li
        liv = jnp.full((L,), li, dtype=jnp.int32)
        cx = plsc.load_gather(x_v, [liv])[0]
        cy = plsc.load_gather(y_v, [liv])[0]
        cz = plsc.load_gather(z_v, [liv])[0]

        lane = _lanes()
        sv = jnp.where(
            lane == 0, m,
            jnp.where(lane == 1, gidx.astype(jnp.float32),
                      jnp.where(lane == 2, cx,
                                jnp.where(lane == 3, cy,
                                          jnp.where(lane == 4, cz,
                                                    jnp.float32(0.0))))))
        stage_v[...] = sv
        parity = lax.rem(s, 2)
        pltpu.sync_copy(stage_v, shared.at[parity, sub])
        plsc.subcore_barrier()
        pltpu.sync_copy(shared.at[parity, pl.ds(gbase, MEMBERS)], gath_v)

        best = gath_v[0, :]
        for k in range(1, MEMBERS):
            cand = gath_v[k, :]
            bd = _ext_f32(best, 0)
            cd = _ext_f32(cand, 0)
            bi = _ext_f32(best, 1)
            ci = _ext_f32(cand, 1)
            take = (cd > bd) | ((cd == bd) & (ci < bi))
            best = jnp.where(take, cand, best)

        wi = _ext_f32(best, 1).astype(jnp.int32)
        wx = _ext_f32(best, 2)
        wy = _ext_f32(best, 3)
        wz = _ext_f32(best, 4)

        lpos = lax.rem(s, L)
        acc_i = jnp.where(lane == lpos, wi, acc_i)
        acc_x = jnp.where(lane == lpos, wx, acc_x)
        acc_y = jnp.where(lane == lpos, wy, acc_y)
        acc_z = jnp.where(lane == lpos, wz, acc_z)

        @pl.when(jnp.logical_and(member == 0, lpos == L - 1))
        def _():
            cbase = (s // L) * L
            idxb_v[pl.ds(cbase, L)] = acc_i
            nxb_v[pl.ds(cbase, L)] = acc_x
            nyb_v[pl.ds(cbase, L)] = acc_y
            nzb_v[pl.ds(cbase, L)] = acc_z

        return (wx, wy, wz, acc_i, acc_x, acc_y, acc_z)

    lax.fori_loop(1, K, step,
                  (px0, py0, pz0, acc0_i, acc0_x, acc0_y, acc0_z))

    @pl.when(member == 0)
    def _():
        pltpu.sync_copy(idxb_v, idx_hbm.at[batch])
        pltpu.sync_copy(nxb_v, nx_hbm.at[batch])
        pltpu.sync_copy(nyb_v, ny_hbm.at[batch])
        pltpu.sync_copy(nzb_v, nz_hbm.at[batch])


def _gather_body(f_hbm, idx_hbm, out_hbm,
                 idx_v, rbase_v, lsel_v, rows_v, orow_v, sem):
    core = lax.axis_index("c")
    sub = lax.axis_index("s")
    w = core * NSUB + sub
    batch = w // (32 // B)
    cbase = (w % (32 // B)) * CPS

    pltpu.sync_copy(idx_hbm.at[batch], idx_v)

    for j in range(IDX_CHUNKS):
        for l in range(8):
            v = idx_v[pl.ds(j * 128 + l * L, L)]
            rbase_v[j, pl.ds(l * L, L)] = lax.shift_right_logical(v, 4)
            lsel_v[pl.ds(j * 128 + l * L, L)] = lax.bitwise_and(v, 15)

    def chan(ci, _):
        c = cbase + ci
        table = f_hbm.at[batch, c]
        copies = [
            pltpu.async_copy(table.at[rbase_v.at[j]],
                             rows_v.at[pl.ds(j * 128, 128)], sem)
            for j in range(IDX_CHUNKS)
        ]
        for cp in copies:
            cp.wait()

        def extract(k, _):
            rowv = k * L + _lanes()
            lanev = lsel_v[pl.ds(k * L, L)]
            orow_v[pl.ds(k * L, L)] = plsc.load_gather(rows_v, [rowv, lanev])
            return 0

        lax.fori_loop(0, K // L, extract, 0)
        pltpu.sync_copy(orow_v, out_hbm.at[batch, c])
        return 0

    lax.fori_loop(0, CPS, chan, 0)


@jax.jit
def kernel(points_xyz, features):
    mesh = plsc.VectorSubcoreMesh(
        core_axis_name="c", subcore_axis_name="s",
        num_cores=NCORE, num_subcores=NSUB)

    x = points_xyz[:, :, 0]
    y = points_xyz[:, :, 1]
    z = points_xyz[:, :, 2]
    p0 = jnp.pad(points_xyz[:, 0, :], ((0, 0), (0, L - 3)))

    fps = pl.kernel(
        _fps_body,
        out_type=(
            jax.ShapeDtypeStruct((B, K), jnp.int32),
            jax.ShapeDtypeStruct((B, K), jnp.float32),
            jax.ShapeDtypeStruct((B, K), jnp.float32),
            jax.ShapeDtypeStruct((B, K), jnp.float32),
        ),
        mesh=mesh,
        scratch_types=[
            pltpu.VMEM((NPER,), jnp.float32),
            pltpu.VMEM((NPER,), jnp.float32),
            pltpu.VMEM((NPER,), jnp.float32),
            pltpu.VMEM((NPER,), jnp.float32),
            pltpu.VMEM((L,), jnp.float32),
            pltpu.VMEM((L,), jnp.float32),
            pltpu.VMEM((MEMBERS, L), jnp.float32),
            pltpu.VMEM((K,), jnp.int32),
            pltpu.VMEM((K,), jnp.float32),
            pltpu.VMEM((K,), jnp.float32),
            pltpu.VMEM((K,), jnp.float32),
            pltpu.VMEM_SHARED((2, NSUB, L), jnp.float32),
        ],
        compiler_params=pltpu.CompilerParams(
            needs_layout_passes=False, use_tc_tiling_on_sc=False),
        name="fps_sc",
    )
    indices, nx, ny, nz = fps(x, y, z, p0)

    f4 = features.reshape(B, C, K, L)
    gather = pl.kernel(
        _gather_body,
        out_type=jax.ShapeDtypeStruct((B, C, K), jnp.float32),
        mesh=mesh,
        scratch_types=[
            pltpu.VMEM((K,), jnp.int32),
            pltpu.VMEM((IDX_CHUNKS, 128), jnp.int32),
            pltpu.VMEM((K,), jnp.int32),
            pltpu.VMEM((K, L), jnp.float32),
            pltpu.VMEM((K,), jnp.float32),
            pltpu.SemaphoreType.DMA,
        ],
        compiler_params=pltpu.CompilerParams(
            needs_layout_passes=False, use_tc_tiling_on_sc=False),
        name="feat_gather_sc",
    )
    new_fea = gather(f4, indices)

    new_xyz = jnp.stack([nx, ny, nz], axis=-1)
    return new_xyz, new_fea, indices


# speedup vs baseline: 2.4489x; 1.0291x over previous
import functools

import jax
import jax.numpy as jnp
from jax import lax
from jax.experimental import pallas as pl
from jax.experimental.pallas import tpu as pltpu
from jax.experimental.pallas import tpu_sc as plsc

B = 8
N = 32768
K = 2048
C = 128
L = 16
NCORE = 2
NSUB = 16
MEMBERS = 4
NPER = N // MEMBERS
NCHUNK = NPER // L
IDX_CHUNKS = K // 128
CPS = C // (32 // B)

def _lanes():
    return lax.iota(jnp.int32, L)


def _ext_f32(v, lane):
    return v[lane]


def _fps_body(x_hbm, y_hbm, z_hbm, p0_hbm,
              idx_hbm, nx_hbm, ny_hbm, nz_hbm,
              x_v, y_v, z_v, d_v, p0_v, stage_v, gath_v,
              idxb_v, nxb_v, nyb_v, nzb_v, shared):
    core = lax.axis_index("c")
    sub = lax.axis_index("s")
    batch = core * (NSUB // MEMBERS) + sub // MEMBERS
    member = sub % MEMBERS
    gbase = (sub // MEMBERS) * MEMBERS
    base = member * NPER

    pltpu.sync_copy(x_hbm.at[batch, pl.ds(base, NPER)], x_v)
    pltpu.sync_copy(y_hbm.at[batch, pl.ds(base, NPER)], y_v)
    pltpu.sync_copy(z_hbm.at[batch, pl.ds(base, NPER)], z_v)
    pltpu.sync_copy(p0_hbm.at[batch], p0_v)

    inf_v = jnp.full((L,), jnp.inf, dtype=jnp.float32)

    def init_chunk(i, _):
        d_v[pl.ds(i * L, L)] = inf_v
        return 0

    lax.fori_loop(0, NCHUNK, init_chunk, 0)

    p0 = p0_v[...]
    px0 = _ext_f32(p0, 0)
    py0 = _ext_f32(p0, 1)
    pz0 = _ext_f32(p0, 2)

    lane = _lanes()
    acc0_i = jnp.zeros((L,), dtype=jnp.int32)
    acc0_x = jnp.where(lane == 0, px0, 0.0)
    acc0_y = jnp.where(lane == 0, py0, 0.0)
    acc0_z = jnp.where(lane == 0, pz0, 0.0)

    UNROLL = 8

    def step(s, carry):
        px, py, pz, acc_i, acc_x, acc_y, acc_z = carry

        def chunk(i, accs):
            outs = []
            for u in range(UNROLL):
                mv, mi = accs[2 * u], accs[2 * u + 1]
                sl = pl.ds((i + u) * L, L)
                dx = x_v[sl] - px
                dy = y_v[sl] - py
                dz = z_v[sl] - pz
                dist = dx * dx + dy * dy + dz * dz
                dn = jnp.minimum(d_v[sl], dist)
                d_v[sl] = dn
                idxs = (i + u) * L + _lanes()
                sel = dn > mv
                outs.append(jnp.where(sel, dn, mv))
                outs.append(jnp.where(sel, idxs, mi))
            return tuple(outs)

        init = (jnp.full((L,), -jnp.inf, dtype=jnp.float32),
                jnp.zeros((L,), dtype=jnp.int32)) * UNROLL
        accs = plsc.parallel_loop(0, NCHUNK, UNROLL, unroll=2,
                                  carry=init)(chunk)

        maxv, maxi = accs[0], accs[1]
        for u in range(1, UNROLL):
            cv, cidx = accs[2 * u], accs[2 * u + 1]
            take = (cv > maxv) | ((cv == maxv) & (cidx < maxi))
            maxv = jnp.where(take, cv, maxv)
            maxi = jnp.where(take, cidx, maxi)

        m = jnp.max(maxv)
        li = jnp.min(jnp.where(maxv == m, maxi, jnp.int32(2147483647)))
        gidx = base + li
        liv = jnp.full((L,), li, dtype=jnp.int32)
        cx = plsc.load_gather(x_v, [liv])[0]
        cy = plsc.load_gather(y_v, [liv])[0]
        cz = plsc.load_gather(z_v, [liv])[0]

        lane = _lanes()
        sv = jnp.where(
            lane == 0, m,
            jnp.where(lane == 1, gidx.astype(jnp.float32),
                      jnp.where(lane == 2, cx,
                                jnp.where(lane == 3, cy,
                                          jnp.where(lane == 4, cz,
                                                    jnp.float32(0.0))))))
        stage_v[...] = sv
        parity = lax.rem(s, 2)
        pltpu.sync_copy(stage_v, shared.at[parity, sub])
        plsc.subcore_barrier()
        pltpu.sync_copy(shared.at[parity, pl.ds(gbase, MEMBERS)], gath_v)

        best = gath_v[0, :]
        for k in range(1, MEMBERS):
            cand = gath_v[k, :]
            bd = _ext_f32(best, 0)
            cd = _ext_f32(cand, 0)
            bi = _ext_f32(best, 1)
            ci = _ext_f32(cand, 1)
            take = (cd > bd) | ((cd == bd) & (ci < bi))
            best = jnp.where(take, cand, best)

        wi = _ext_f32(best, 1).astype(jnp.int32)
        wx = _ext_f32(best, 2)
        wy = _ext_f32(best, 3)
        wz = _ext_f32(best, 4)

        lpos = lax.rem(s, L)
        acc_i = jnp.where(lane == lpos, wi, acc_i)
        acc_x = jnp.where(lane == lpos, wx, acc_x)
        acc_y = jnp.where(lane == lpos, wy, acc_y)
        acc_z = jnp.where(lane == lpos, wz, acc_z)

        @pl.when(jnp.logical_and(member == 0, lpos == L - 1))
        def _():
            cbase = (s // L) * L
            idxb_v[pl.ds(cbase, L)] = acc_i
            nxb_v[pl.ds(cbase, L)] = acc_x
            nyb_v[pl.ds(cbase, L)] = acc_y
            nzb_v[pl.ds(cbase, L)] = acc_z

        return (wx, wy, wz, acc_i, acc_x, acc_y, acc_z)

    lax.fori_loop(1, K, step,
                  (px0, py0, pz0, acc0_i, acc0_x, acc0_y, acc0_z))

    @pl.when(member == 0)
    def _():
        pltpu.sync_copy(idxb_v, idx_hbm.at[batch])
        pltpu.sync_copy(nxb_v, nx_hbm.at[batch])
        pltpu.sync_copy(nyb_v, ny_hbm.at[batch])
        pltpu.sync_copy(nzb_v, nz_hbm.at[batch])


def _gather_body(f_hbm, idx_hbm, out_hbm,
                 idx_v, rbase_v, lsel_v, rows_v, orow_v, sem):
    core = lax.axis_index("c")
    sub = lax.axis_index("s")
    w = core * NSUB + sub
    batch = w // (32 // B)
    cbase = (w % (32 // B)) * CPS

    pltpu.sync_copy(idx_hbm.at[batch], idx_v)

    for j in range(IDX_CHUNKS):
        for l in range(8):
            v = idx_v[pl.ds(j * 128 + l * L, L)]
            rbase_v[j, pl.ds(l * L, L)] = lax.shift_right_logical(v, 4)
            lsel_v[pl.ds(j * 128 + l * L, L)] = lax.bitwise_and(v, 15)

    def chan(ci, _):
        c = cbase + ci
        table = f_hbm.at[batch, c]
        copies = [
            pltpu.async_copy(table.at[rbase_v.at[j]],
                             rows_v.at[pl.ds(j * 128, 128)], sem)
            for j in range(IDX_CHUNKS)
        ]
        for cp in copies:
            cp.wait()

        def extract(k, _):
            rowv = k * L + _lanes()
            lanev = lsel_v[pl.ds(k * L, L)]
            orow_v[pl.ds(k * L, L)] = plsc.load_gather(rows_v, [rowv, lanev])
            return 0

        lax.fori_loop(0, K // L, extract, 0)
        pltpu.sync_copy(orow_v, out_hbm.at[batch, c])
        return 0

    lax.fori_loop(0, CPS, chan, 0)


@jax.jit
def kernel(points_xyz, features):
    mesh = plsc.VectorSubcoreMesh(
        core_axis_name="c", subcore_axis_name="s",
        num_cores=NCORE, num_subcores=NSUB)

    x = points_xyz[:, :, 0]
    y = points_xyz[:, :, 1]
    z = points_xyz[:, :, 2]
    p0 = jnp.pad(points_xyz[:, 0, :], ((0, 0), (0, L - 3)))

    fps = pl.kernel(
        _fps_body,
        out_type=(
            jax.ShapeDtypeStruct((B, K), jnp.int32),
            jax.ShapeDtypeStruct((B, K), jnp.float32),
            jax.ShapeDtypeStruct((B, K), jnp.float32),
            jax.ShapeDtypeStruct((B, K), jnp.float32),
        ),
        mesh=mesh,
        scratch_types=[
            pltpu.VMEM((NPER,), jnp.float32),
            pltpu.VMEM((NPER,), jnp.float32),
            pltpu.VMEM((NPER,), jnp.float32),
            pltpu.VMEM((NPER,), jnp.float32),
            pltpu.VMEM((L,), jnp.float32),
            pltpu.VMEM((L,), jnp.float32),
            pltpu.VMEM((MEMBERS, L), jnp.float32),
            pltpu.VMEM((K,), jnp.int32),
            pltpu.VMEM((K,), jnp.float32),
            pltpu.VMEM((K,), jnp.float32),
            pltpu.VMEM((K,), jnp.float32),
            pltpu.VMEM_SHARED((2, NSUB, L), jnp.float32),
        ],
        compiler_params=pltpu.CompilerParams(
            needs_layout_passes=False, use_tc_tiling_on_sc=False),
        name="fps_sc",
    )
    indices, nx, ny, nz = fps(x, y, z, p0)

    f4 = features.reshape(B, C, K, L)
    gather = pl.kernel(
        _gather_body,
        out_type=jax.ShapeDtypeStruct((B, C, K), jnp.float32),
        mesh=mesh,
        scratch_types=[
            pltpu.VMEM((K,), jnp.int32),
            pltpu.VMEM((IDX_CHUNKS, 128), jnp.int32),
            pltpu.VMEM((K,), jnp.int32),
            pltpu.VMEM((K, L), jnp.float32),
            pltpu.VMEM((K,), jnp.float32),
            pltpu.SemaphoreType.DMA,
        ],
        compiler_params=pltpu.CompilerParams(
            needs_layout_passes=False, use_tc_tiling_on_sc=False),
        name="feat_gather_sc",
    )
    new_fea = gather(f4, indices)

    new_xyz = jnp.stack([nx, ny, nz], axis=-1)
    return new_xyz, new_fea, indices


# speedup vs baseline: 2.8817x; 1.1767x over previous
import functools

import jax
import jax.numpy as jnp
from jax import lax
from jax.experimental import pallas as pl
from jax.experimental.pallas import tpu as pltpu
from jax.experimental.pallas import tpu_sc as plsc

B = 8
N = 32768
K = 2048
C = 128
L = 16
NCORE = 2
NSUB = 16
MEMBERS = 4
NPER = N // MEMBERS
NCHUNK = NPER // L
IDX_CHUNKS = K // 128
CPS = C // (32 // B)

def _lanes():
    return lax.iota(jnp.int32, L)


def _ext_f32(v, lane):
    return v[lane]


def _fps_body(x_hbm, y_hbm, z_hbm, p0_hbm,
              idx_hbm, nx_hbm, ny_hbm, nz_hbm,
              x_v, y_v, z_v, d_v, p0_v, stage_v, gath_v,
              idxb_v, nxb_v, nyb_v, nzb_v, shared):
    core = lax.axis_index("c")
    sub = lax.axis_index("s")
    batch = core * (NSUB // MEMBERS) + sub // MEMBERS
    member = sub % MEMBERS
    gbase = (sub // MEMBERS) * MEMBERS
    base = member * NPER

    pltpu.sync_copy(x_hbm.at[batch, pl.ds(base, NPER)], x_v)
    pltpu.sync_copy(y_hbm.at[batch, pl.ds(base, NPER)], y_v)
    pltpu.sync_copy(z_hbm.at[batch, pl.ds(base, NPER)], z_v)
    pltpu.sync_copy(p0_hbm.at[batch], p0_v)

    inf_v = jnp.full((L,), jnp.inf, dtype=jnp.float32)

    def init_chunk(i, _):
        d_v[pl.ds(i * L, L)] = inf_v
        return 0

    lax.fori_loop(0, NCHUNK, init_chunk, 0)

    p0 = p0_v[...]
    px0 = _ext_f32(p0, 0)
    py0 = _ext_f32(p0, 1)
    pz0 = _ext_f32(p0, 2)

    lane = _lanes()
    acc0_i = jnp.zeros((L,), dtype=jnp.int32)
    acc0_x = jnp.where(lane == 0, px0, 0.0)
    acc0_y = jnp.where(lane == 0, py0, 0.0)
    acc0_z = jnp.where(lane == 0, pz0, 0.0)

    UNROLL = 8

    def step(s, carry):
        px, py, pz, acc_i, acc_x, acc_y, acc_z = carry

        def chunk(i, accs):
            outs = []
            for u in range(UNROLL):
                mv, mi = accs[2 * u], accs[2 * u + 1]
                sl = pl.ds((i + u) * L, L)
                dx = x_v[sl] - px
                dy = y_v[sl] - py
                dz = z_v[sl] - pz
                dist = dx * dx + dy * dy + dz * dz
                dn = jnp.minimum(d_v[sl], dist)
                d_v[sl] = dn
                idxs = (i + u) * L + _lanes()
                sel = dn > mv
                outs.append(jnp.where(sel, dn, mv))
                outs.append(jnp.where(sel, idxs, mi))
            return tuple(outs)

        init = (jnp.full((L,), -jnp.inf, dtype=jnp.float32),
                jnp.zeros((L,), dtype=jnp.int32)) * UNROLL
        accs = plsc.parallel_loop(0, NCHUNK, UNROLL, unroll=1,
                                  carry=init)(chunk)

        maxv, maxi = accs[0], accs[1]
        for u in range(1, UNROLL):
            cv, cidx = accs[2 * u], accs[2 * u + 1]
            take = (cv > maxv) | ((cv == maxv) & (cidx < maxi))
            maxv = jnp.where(take, cv, maxv)
            maxi = jnp.where(take, cidx, maxi)

        m = jnp.max(maxv)
        li = jnp.min(jnp.where(maxv == m, maxi, jnp.int32(2147483647)))
        gidx = base + li
        liv = jnp.full((L,), li, dtype=jnp.int32)
        cx = plsc.load_gather(x_v, [liv])[0]
        cy = plsc.load_gather(y_v, [liv])[0]
        cz = plsc.load_gather(z_v, [liv])[0]

        lane = _lanes()
        sv = jnp.where(
            lane == 0, m,
            jnp.where(lane == 1, gidx.astype(jnp.float32),
                      jnp.where(lane == 2, cx,
                                jnp.where(lane == 3, cy,
                                          jnp.where(lane == 4, cz,
                                                    jnp.float32(0.0))))))
        stage_v[...] = sv
        parity = lax.rem(s, 2)
        pltpu.sync_copy(stage_v, shared.at[parity, sub])
        plsc.subcore_barrier()
        pltpu.sync_copy(shared.at[parity, pl.ds(gbase, MEMBERS)], gath_v)

        best = gath_v[0, :]
        for k in range(1, MEMBERS):
            cand = gath_v[k, :]
            bd = _ext_f32(best, 0)
            cd = _ext_f32(cand, 0)
            bi = _ext_f32(best, 1)
            ci = _ext_f32(cand, 1)
            take = (cd > bd) | ((cd == bd) & (ci < bi))
            best = jnp.where(take, cand, best)

        wi = _ext_f32(best, 1).astype(jnp.int32)
        wx = _ext_f32(best, 2)
        wy = _ext_f32(best, 3)
        wz = _ext_f32(best, 4)

        lpos = lax.rem(s, L)
        acc_i = jnp.where(lane == lpos, wi, acc_i)
        acc_x = jnp.where(lane == lpos, wx, acc_x)
        acc_y = jnp.where(lane == lpos, wy, acc_y)
        acc_z = jnp.where(lane == lpos, wz, acc_z)

        @pl.when(jnp.logical_and(member == 0, lpos == L - 1))
        def _():
            cbase = (s // L) * L
            idxb_v[pl.ds(cbase, L)] = acc_i
            nxb_v[pl.ds(cbase, L)] = acc_x
            nyb_v[pl.ds(cbase, L)] = acc_y
            nzb_v[pl.ds(cbase, L)] = acc_z

        return (wx, wy, wz, acc_i, acc_x, acc_y, acc_z)

    lax.fori_loop(1, K, step,
                  (px0, py0, pz0, acc0_i, acc0_x, acc0_y, acc0_z))

    @pl.when(member == 0)
    def _():
        pltpu.sync_copy(idxb_v, idx_hbm.at[batch])
        pltpu.sync_copy(nxb_v, nx_hbm.at[batch])
        pltpu.sync_copy(nyb_v, ny_hbm.at[batch])
        pltpu.sync_copy(nzb_v, nz_hbm.at[batch])


def _gather_body(f_hbm, idx_hbm, out_hbm,
                 idx_v, rbase_v, lsel_v, rows_v, orow_v, sem):
    core = lax.axis_index("c")
    sub = lax.axis_index("s")
    w = core * NSUB + sub
    batch = w // (32 // B)
    cbase = (w % (32 // B)) * CPS

    pltpu.sync_copy(idx_hbm.at[batch], idx_v)

    for j in range(IDX_CHUNKS):
        for l in range(8):
            v = idx_v[pl.ds(j * 128 + l * L, L)]
            rbase_v[j, pl.ds(l * L, L)] = lax.shift_right_logical(v, 4)
            lsel_v[pl.ds(j * 128 + l * L, L)] = lax.bitwise_and(v, 15)

    def chan(ci, _):
        c = cbase + ci
        table = f_hbm.at[batch, c]
        copies = [
            pltpu.async_copy(table.at[rbase_v.at[j]],
                             rows_v.at[pl.ds(j * 128, 128)], sem)
            for j in range(IDX_CHUNKS)
        ]
        for cp in copies:
            cp.wait()

        def extract(k, _):
            rowv = k * L + _lanes()
            lanev = lsel_v[pl.ds(k * L, L)]
            orow_v[pl.ds(k * L, L)] = plsc.load_gather(rows_v, [rowv, lanev])
            return 0

        lax.fori_loop(0, K // L, extract, 0)
        pltpu.sync_copy(orow_v, out_hbm.at[batch, c])
        return 0

    lax.fori_loop(0, CPS, chan, 0)


@jax.jit
def kernel(points_xyz, features):
    mesh = plsc.VectorSubcoreMesh(
        core_axis_name="c", subcore_axis_name="s",
        num_cores=NCORE, num_subcores=NSUB)

    x = points_xyz[:, :, 0]
    y = points_xyz[:, :, 1]
    z = points_xyz[:, :, 2]
    p0 = jnp.pad(points_xyz[:, 0, :], ((0, 0), (0, L - 3)))

    fps = pl.kernel(
        _fps_body,
        out_type=(
            jax.ShapeDtypeStruct((B, K), jnp.int32),
            jax.ShapeDtypeStruct((B, K), jnp.float32),
            jax.ShapeDtypeStruct((B, K), jnp.float32),
            jax.ShapeDtypeStruct((B, K), jnp.float32),
        ),
        mesh=mesh,
        scratch_types=[
            pltpu.VMEM((NPER,), jnp.float32),
            pltpu.VMEM((NPER,), jnp.float32),
            pltpu.VMEM((NPER,), jnp.float32),
            pltpu.VMEM((NPER,), jnp.float32),
            pltpu.VMEM((L,), jnp.float32),
            pltpu.VMEM((L,), jnp.float32),
            pltpu.VMEM((MEMBERS, L), jnp.float32),
            pltpu.VMEM((K,), jnp.int32),
            pltpu.VMEM((K,), jnp.float32),
            pltpu.VMEM((K,), jnp.float32),
            pltpu.VMEM((K,), jnp.float32),
            pltpu.VMEM_SHARED((2, NSUB, L), jnp.float32),
        ],
        compiler_params=pltpu.CompilerParams(
            needs_layout_passes=False, use_tc_tiling_on_sc=False),
        name="fps_sc",
    )
    indices, nx, ny, nz = fps(x, y, z, p0)

    f4 = features.reshape(B, C, K, L)
    gather = pl.kernel(
        _gather_body,
        out_type=jax.ShapeDtypeStruct((B, C, K), jnp.float32),
        mesh=mesh,
        scratch_types=[
            pltpu.VMEM((K,), jnp.int32),
            pltpu.VMEM((IDX_CHUNKS, 128), jnp.int32),
            pltpu.VMEM((K,), jnp.int32),
            pltpu.VMEM((K, L), jnp.float32),
            pltpu.VMEM((K,), jnp.float32),
            pltpu.SemaphoreType.DMA,
        ],
        compiler_params=pltpu.CompilerParams(
            needs_layout_passes=False, use_tc_tiling_on_sc=False),
        name="feat_gather_sc",
    )
    new_fea = gather(f4, indices)

    new_xyz = jnp.stack([nx, ny, nz], axis=-1)
    return new_xyz, new_fea, indices
